# baseline (speedup 1.0000x reference)
import functools

import jax
import jax.numpy as jnp
from jax import lax
from jax.experimental import pallas as pl
from jax.experimental.pallas import tpu as pltpu
from jax.experimental.pallas import tpu_sc as plsc

N = 10000
NPAD = 10240
E = 640000
NW = 32
CH = 128
NCH = 160
EPAD = NW * NCH * CH
NCHT = EPAD // CH
N0 = 160
N1 = 160
ROWS = NPAD // 16
BLK = 32

def _mesh():
    return plsc.VectorSubcoreMesh(core_axis_name="c", subcore_axis_name="s")


_TC_PARAMS = pltpu.CompilerParams(vmem_limit_bytes=100 * 1024 * 1024)


def _deg_call(dstp, ones128, z128):
    @functools.partial(
        pl.kernel,
        out_type=jax.ShapeDtypeStruct((2, NPAD, 128), jnp.float32),
        mesh=_mesh(),
        scratch_types=[
            pltpu.VMEM((NCH, CH), jnp.int32),
            pltpu.VMEM((CH, 128), jnp.float32),
            pltpu.VMEM_SHARED((NPAD, 128), jnp.float32),
        ],
    )
    def k(dst_hbm, ones_hbm, z_hbm, out_hbm, dst_v, ones_v, acc):
        c = lax.axis_index("c")
        s = lax.axis_index("s")
        wid = c * 16 + s
        r0 = s * ROWS
        pltpu.sync_copy(z_hbm.at[pl.ds(r0, ROWS)], acc.at[pl.ds(r0, ROWS)])
        pltpu.sync_copy(dst_hbm.at[pl.ds(wid * NCH, NCH)], dst_v)
        pltpu.sync_copy(ones_hbm, ones_v)
        plsc.subcore_barrier()

        def body(j, carry):
            pltpu.sync_copy(ones_v, acc.at[dst_v.at[j]], add=True)
            return carry

        lax.fori_loop(0, NCH, body, 0)
        plsc.subcore_barrier()
        pltpu.sync_copy(acc.at[pl.ds(r0, ROWS)], out_hbm.at[c, pl.ds(r0, ROWS)])

    return k(dstp, ones128, z128)


def _agg_call(hs, srcp, dstp, zeros):
    F = hs.shape[1]

    NBUF = 2

    @functools.partial(
        pl.kernel,
        out_type=jax.ShapeDtypeStruct((2, NPAD, F), jnp.float32),
        mesh=_mesh(),
        scratch_types=[
            pltpu.VMEM((BLK, CH), jnp.int32),
            pltpu.VMEM((BLK, CH), jnp.int32),
        ] + [pltpu.VMEM((CH, F), jnp.float32)] * NBUF + [
            pltpu.VMEM_SHARED((NPAD, F), jnp.float32),
        ] + [pltpu.SemaphoreType.DMA] * NBUF,
    )
    def k(hs_hbm, src_hbm, dst_hbm, z_hbm, out_hbm, src_v, dst_v, *rest):
        bufs = rest[:NBUF]
        acc = rest[NBUF]
        sems = rest[NBUF + 1:]
        c = lax.axis_index("c")
        s = lax.axis_index("s")
        r0 = s * ROWS
        base = jnp.where(c == 0, s * N0, 16 * N0 + s * N1)
        nblk = jnp.where(c == 0, N0 // BLK, N1 // BLK)
        pltpu.sync_copy(z_hbm.at[pl.ds(r0, ROWS)], acc.at[pl.ds(r0, ROWS)])
        plsc.subcore_barrier()

        def outer(bi, carry):
            pltpu.sync_copy(src_hbm.at[pl.ds(base + bi * BLK, BLK)], src_v)
            pltpu.sync_copy(dst_hbm.at[pl.ds(base + bi * BLK, BLK)], dst_v)

            for b in range(NBUF):
                pltpu.async_copy(hs_hbm.at[src_v.at[b]], bufs[b], sems[b])

            def body(g, c2):
                for b in range(NBUF):
                    j = g * NBUF + b
                    pltpu.make_async_copy(hs_hbm.at[src_v.at[j]], bufs[b],
                                          sems[b]).wait()
                    pltpu.sync_copy(bufs[b], acc.at[dst_v.at[j]], add=True)

                    @pl.when(j + NBUF < BLK)
                    def _():
                        pltpu.async_copy(hs_hbm.at[src_v.at[j + NBUF]],
                                         bufs[b], sems[b])
                return c2

            return lax.fori_loop(0, BLK // NBUF, body, carry)

        lax.fori_loop(0, nblk, outer, 0)
        plsc.subcore_barrier()
        pltpu.sync_copy(acc.at[pl.ds(r0, ROWS)], out_hbm.at[c, pl.ds(r0, ROWS)])

    return k(hs, srcp, dstp, zeros)


def _dot(a, b):
    return jnp.dot(a, b, preferred_element_type=jnp.float32)


def _row_mask():
    return lax.broadcasted_iota(jnp.int32, (NPAD, 1), 0) < N


def _elu(z):
    return jnp.where(z > 0, z, jnp.exp(jnp.minimum(z, 0.0)) - 1.0)


def _tc_a(xp, W0p, degp):
    def body(x_ref, w_ref, degp_ref, hs_ref, dinv_ref):
        d3 = degp_ref[...]
        deg = d3[0, :, 0:1] + d3[1, :, 0:1] + 1.0
        dinv = lax.rsqrt(deg)
        hw = jnp.dot(x_ref[...], w_ref[...],
                     preferred_element_type=jnp.float32,
                     precision=lax.Precision.HIGHEST)
        hs_ref[...] = hw * dinv
        dinv_ref[...] = dinv

    return pl.pallas_call(
        body,
        out_shape=[
            jax.ShapeDtypeStruct((NPAD, 128), jnp.float32),
            jax.ShapeDtypeStruct((NPAD, 1), jnp.float32),
        ],
    )(xp, W0p, degp)


def _tc_combine(p, hs_cur, dinv, b, g, be, W_next=None, h_res=None):
    outs = [jax.ShapeDtypeStruct((NPAD, 128), jnp.float32)]
    if W_next is not None:
        outs.append(jax.ShapeDtypeStruct((NPAD, 128), jnp.float32))

    def body(*refs):
        p_ref, hs_ref, dinv_ref, b_ref, g_ref, be_ref = refs[:6]
        idx = 6
        w_ref = res_ref = None
        if W_next is not None:
            w_ref = refs[idx]; idx += 1
        if h_res is not None:
            res_ref = refs[idx]; idx += 1
        h_ref = refs[idx]; idx += 1
        hsn_ref = refs[idx] if W_next is not None else None

        dinv = dinv_ref[...]
        pp = p_ref[...]
        y = dinv * (pp[0] + pp[1] + hs_ref[...]) + b_ref[...][None, :]
        mask = _row_mask()
        ym = jnp.where(mask, y, 0.0)
        mean = jnp.sum(ym, axis=0, keepdims=True) * (1.0 / N)
        dy = jnp.where(mask, y - mean, 0.0)
        var = jnp.sum(dy * dy, axis=0, keepdims=True) * (1.0 / N)
        z = (y - mean) * lax.rsqrt(var + 1e-5) * g_ref[...][None, :] \
            + be_ref[...][None, :]
        if h_res is not None:
            z = z + res_ref[...]
        h = jnp.where(mask, _elu(z), 0.0)
        h_ref[...] = h
        if W_next is not None:
            hsn_ref[...] = _dot(h, w_ref[...]) * dinv

    args = [p, hs_cur, dinv, b, g, be]
    if W_next is not None:
        args.append(W_next)
    if h_res is not None:
        args.append(h_res)
    res = pl.pallas_call(body, out_shape=outs, compiler_params=_TC_PARAMS)(*args)
    return res


def _tc_head(h0, h1, h2, h3, att_wT, fc1_W, ln1_g, ln1_b, fc2_W, ln2_g, ln2_b,
             fc3_W, ln3_g, ln3_b, fco_W, fco_b):
    def ln(z, g_ref, b_ref):
        m = jnp.mean(z, axis=1, keepdims=True)
        d = z - m
        v = jnp.mean(d * d, axis=1, keepdims=True)
        return d * lax.rsqrt(v + 1e-5) * g_ref[...][None, :] \
            + b_ref[...][None, :]

    def body(h0r, h1r, h2r, h3r, awr, f1r, g1r, b1r, f2r, g2r, b2r,
             f3r, g3r, b3r, fwr, fbr, outr):
        f = jnp.concatenate(
            [h0r[...][:, :64], h1r[...], h2r[...], h3r[...][:, :64]],
            axis=1)
        logits = _dot(f, awr[...])[:, 0:1]
        mask = _row_mask()
        m = jnp.max(jnp.where(mask, logits, -jnp.inf), axis=0, keepdims=True)
        e = jnp.where(mask, jnp.exp(logits - m), 0.0)
        aw = e / jnp.sum(e, axis=0, keepdims=True)
        a = f * aw
        amax = jnp.max(jnp.where(mask, a, -jnp.inf), axis=0, keepdims=True)
        asum = jnp.sum(jnp.where(mask, a, 0.0), axis=0, keepdims=True)
        pooled = jnp.concatenate([amax, asum * (1.0 / N), asum], axis=1)
        hidot = functools.partial(jnp.dot,
                                  preferred_element_type=jnp.float32,
                                  precision=lax.Precision.HIGHEST)
        z = jnp.broadcast_to(pooled, (8, 3 * 384))
        z = _elu(ln(hidot(z, f1r[...]), g1r, b1r))
        z = _elu(ln(hidot(z, f2r[...]), g2r, b2r))
        z = _elu(ln(hidot(z, f3r[...]), g3r, b3r))
        z = hidot(z, fwr[...]) + fbr[...][None, :]
        outr[...] = z[0:1, :]

    return pl.pallas_call(
        body,
        out_shape=jax.ShapeDtypeStruct((1, 6), jnp.float32),
        compiler_params=_TC_PARAMS,
    )(h0, h1, h2, h3, att_wT, fc1_W, ln1_g, ln1_b, fc2_W, ln2_g, ln2_b,
      fc3_W, ln3_g, ln3_b, fco_W, fco_b)


def kernel(x, adj_or_data, W0, b0, g0, be0, W1, b1, g1, be1, W2, b2, g2, be2,
           W3, b3, g3, be3, att_w, fc1_W, ln1_g, ln1_b, fc2_W, ln2_g, ln2_b,
           fc3_W, ln3_g, ln3_b, fco_W, fco_b):
    src = adj_or_data[0]
    dst = adj_or_data[1]
    padi = jnp.full((EPAD - E,), N, dtype=jnp.int32)
    srcp = jnp.concatenate([src, padi]).reshape(NCHT, CH)
    dstp = jnp.concatenate([dst, padi]).reshape(NCHT, CH)
    ones128 = jnp.ones((CH, 128), jnp.float32)
    z128 = jnp.zeros((NPAD, 128), jnp.float32)
    xp = jnp.pad(x, ((0, NPAD - N), (0, 5)))
    W0p = jnp.pad(W0, ((0, 5), (0, 64)))
    W1p = jnp.pad(W1, ((0, 64), (0, 0)))
    W3p = jnp.pad(W3, ((0, 0), (0, 64)))
    pad64 = lambda v: jnp.pad(v, (0, 64))

    degp = _deg_call(dstp, ones128, z128)
    hs1, dinv = _tc_a(xp, W0p, degp)

    p1 = _agg_call(hs1, srcp, dstp, z128)
    h0, hs2 = _tc_combine(p1, hs1, dinv, pad64(b0), pad64(g0), pad64(be0),
                          W_next=W1p)

    p2 = _agg_call(hs2, srcp, dstp, z128)
    h1, hs3 = _tc_combine(p2, hs2, dinv, b1, g1, be1, W_next=W2)

    p3 = _agg_call(hs3, srcp, dstp, z128)
    h2, hs4 = _tc_combine(p3, hs3, dinv, b2, g2, be2, W_next=W3p, h_res=h1)

    p4 = _agg_call(hs4, srcp, dstp, z128)
    (h3,) = _tc_combine(p4, hs4, dinv, pad64(b3), pad64(g3), pad64(be3))

    return _tc_head(h0, h1, h2, h3, jnp.pad(att_w, ((0, 0), (0, 127))), fc1_W,
                    ln1_g, ln1_b, fc2_W, ln2_g, ln2_b,
                    fc3_W, ln3_g, ln3_b, fco_W, fco_b)


# speedup vs baseline: 1.1762x; 1.1762x over previous
import functools

import jax
import jax.numpy as jnp
from jax import lax
from jax.experimental import pallas as pl
from jax.experimental.pallas import tpu as pltpu
from jax.experimental.pallas import tpu_sc as plsc

N = 10000
NPAD = 10240
E = 640000
NW = 32
CH = 128
NCH = 160
EPAD = NW * NCH * CH
ROWS = NPAD // 16
BLK = 32

def _mesh():
    return plsc.VectorSubcoreMesh(core_axis_name="c", subcore_axis_name="s")


_TC_PARAMS = pltpu.CompilerParams(vmem_limit_bytes=100 * 1024 * 1024)


def _deg_call(dstp, ones128, z128):
    @functools.partial(
        pl.kernel,
        out_type=jax.ShapeDtypeStruct((2, NPAD, 128), jnp.float32),
        mesh=_mesh(),
        scratch_types=[
            pltpu.VMEM((NCH, CH), jnp.int32),
            pltpu.VMEM((CH, 128), jnp.float32),
            pltpu.VMEM_SHARED((NPAD, 128), jnp.float32),
        ],
    )
    def k(dst_hbm, ones_hbm, z_hbm, out_hbm, dst_v, ones_v, acc):
        c = lax.axis_index("c")
        s = lax.axis_index("s")
        wid = c * 16 + s
        r0 = s * ROWS
        pltpu.sync_copy(z_hbm.at[pl.ds(r0, ROWS)], acc.at[pl.ds(r0, ROWS)])
        pltpu.sync_copy(dst_hbm.at[wid], dst_v)
        pltpu.sync_copy(ones_hbm, ones_v)
        plsc.subcore_barrier()

        def body(j, carry):
            pltpu.sync_copy(ones_v, acc.at[dst_v.at[j]], add=True)
            return carry

        lax.fori_loop(0, NCH, body, 0)
        plsc.subcore_barrier()
        pltpu.sync_copy(acc.at[pl.ds(r0, ROWS)], out_hbm.at[c, pl.ds(r0, ROWS)])

    return k(dstp, ones128, z128)


def _agg_call(hs, srcp, dstp, zeros):
    F = hs.shape[1]

    NBUF = 2

    @functools.partial(
        pl.kernel,
        out_type=jax.ShapeDtypeStruct((2, NPAD, F), jnp.float32),
        mesh=_mesh(),
        scratch_types=[
            pltpu.VMEM((BLK, CH), jnp.int32),
            pltpu.VMEM((BLK, CH), jnp.int32),
        ] + [pltpu.VMEM((CH, F), jnp.float32)] * NBUF + [
            pltpu.VMEM_SHARED((NPAD, F), jnp.float32),
        ] + [pltpu.SemaphoreType.DMA] * NBUF,
    )
    def k(hs_hbm, src_hbm, dst_hbm, z_hbm, out_hbm, src_v, dst_v, *rest):
        bufs = rest[:NBUF]
        acc = rest[NBUF]
        sems = rest[NBUF + 1:]
        c = lax.axis_index("c")
        s = lax.axis_index("s")
        wid = c * 16 + s
        r0 = s * ROWS
        pltpu.sync_copy(z_hbm.at[pl.ds(r0, ROWS)], acc.at[pl.ds(r0, ROWS)])
        plsc.subcore_barrier()

        def outer(bi, carry):
            pltpu.sync_copy(src_hbm.at[wid, pl.ds(bi * BLK, BLK)], src_v)
            pltpu.sync_copy(dst_hbm.at[wid, pl.ds(bi * BLK, BLK)], dst_v)

            for b in range(NBUF):
                pltpu.async_copy(hs_hbm.at[src_v.at[b]], bufs[b], sems[b])

            def body(g, c2):
                for b in range(NBUF):
                    j = g * NBUF + b
                    pltpu.make_async_copy(hs_hbm.at[src_v.at[j]], bufs[b],
                                          sems[b]).wait()
                    pltpu.sync_copy(bufs[b], acc.at[dst_v.at[j]], add=True)

                    @pl.when(j + NBUF < BLK)
                    def _():
                        pltpu.async_copy(hs_hbm.at[src_v.at[j + NBUF]],
                                         bufs[b], sems[b])
                return c2

            return lax.fori_loop(0, BLK // NBUF, body, carry)

        lax.fori_loop(0, NCH // BLK, outer, 0)
        plsc.subcore_barrier()
        pltpu.sync_copy(acc.at[pl.ds(r0, ROWS)], out_hbm.at[c, pl.ds(r0, ROWS)])

    return k(hs, srcp, dstp, zeros)


def _dot(a, b):
    return jnp.dot(a, b, preferred_element_type=jnp.float32)


def _row_mask():
    return lax.broadcasted_iota(jnp.int32, (NPAD, 1), 0) < N


def _elu(z):
    return jnp.where(z > 0, z, jnp.exp(jnp.minimum(z, 0.0)) - 1.0)


def _tc_a(xp, W0p, degp):
    def body(x_ref, w_ref, degp_ref, hs_ref, dinv_ref):
        d3 = degp_ref[...]
        deg = d3[0, :, 0:1] + d3[1, :, 0:1] + 1.0
        dinv = lax.rsqrt(deg)
        hw = jnp.dot(x_ref[...], w_ref[...],
                     preferred_element_type=jnp.float32,
                     precision=lax.Precision.HIGHEST)
        hs_ref[...] = hw * dinv
        dinv_ref[...] = dinv

    return pl.pallas_call(
        body,
        out_shape=[
            jax.ShapeDtypeStruct((NPAD, 128), jnp.float32),
            jax.ShapeDtypeStruct((NPAD, 1), jnp.float32),
        ],
    )(xp, W0p, degp)


def _tc_combine(p, hs_cur, dinv, b, g, be, W_next=None, h_res=None):
    outs = [jax.ShapeDtypeStruct((NPAD, 128), jnp.float32)]
    if W_next is not None:
        outs.append(jax.ShapeDtypeStruct((NPAD, 128), jnp.float32))

    def body(*refs):
        p_ref, hs_ref, dinv_ref, b_ref, g_ref, be_ref = refs[:6]
        idx = 6
        w_ref = res_ref = None
        if W_next is not None:
            w_ref = refs[idx]; idx += 1
        if h_res is not None:
            res_ref = refs[idx]; idx += 1
        h_ref = refs[idx]; idx += 1
        hsn_ref = refs[idx] if W_next is not None else None

        dinv = dinv_ref[...]
        pp = p_ref[...]
        y = dinv * (pp[0] + pp[1] + hs_ref[...]) + b_ref[...][None, :]
        mask = _row_mask()
        ym = jnp.where(mask, y, 0.0)
        mean = jnp.sum(ym, axis=0, keepdims=True) * (1.0 / N)
        dy = jnp.where(mask, y - mean, 0.0)
        var = jnp.sum(dy * dy, axis=0, keepdims=True) * (1.0 / N)
        z = (y - mean) * lax.rsqrt(var + 1e-5) * g_ref[...][None, :] \
            + be_ref[...][None, :]
        if h_res is not None:
            z = z + res_ref[...]
        h = jnp.where(mask, _elu(z), 0.0)
        h_ref[...] = h
        if W_next is not None:
            hsn_ref[...] = _dot(h, w_ref[...]) * dinv

    args = [p, hs_cur, dinv, b, g, be]
    if W_next is not None:
        args.append(W_next)
    if h_res is not None:
        args.append(h_res)
    res = pl.pallas_call(body, out_shape=outs, compiler_params=_TC_PARAMS)(*args)
    return res


def _tc_head(h0, h1, h2, h3, att_wT, fc1_W, ln1_g, ln1_b, fc2_W, ln2_g, ln2_b,
             fc3_W, ln3_g, ln3_b, fco_W, fco_b):
    def ln(z, g_ref, b_ref):
        m = jnp.mean(z, axis=1, keepdims=True)
        d = z - m
        v = jnp.mean(d * d, axis=1, keepdims=True)
        return d * lax.rsqrt(v + 1e-5) * g_ref[...][None, :] \
            + b_ref[...][None, :]

    def body(h0r, h1r, h2r, h3r, awr, f1r, g1r, b1r, f2r, g2r, b2r,
             f3r, g3r, b3r, fwr, fbr, outr):
        f = jnp.concatenate(
            [h0r[...][:, :64], h1r[...], h2r[...], h3r[...][:, :64]],
            axis=1)
        logits = _dot(f, awr[...])[:, 0:1]
        mask = _row_mask()
        m = jnp.max(jnp.where(mask, logits, -jnp.inf), axis=0, keepdims=True)
        e = jnp.where(mask, jnp.exp(logits - m), 0.0)
        aw = e / jnp.sum(e, axis=0, keepdims=True)
        a = f * aw
        amax = jnp.max(jnp.where(mask, a, -jnp.inf), axis=0, keepdims=True)
        asum = jnp.sum(jnp.where(mask, a, 0.0), axis=0, keepdims=True)
        pooled = jnp.concatenate([amax, asum * (1.0 / N), asum], axis=1)
        hidot = functools.partial(jnp.dot,
                                  preferred_element_type=jnp.float32,
                                  precision=lax.Precision.HIGHEST)
        z = jnp.broadcast_to(pooled, (8, 3 * 384))
        z = _elu(ln(hidot(z, f1r[...]), g1r, b1r))
        z = _elu(ln(hidot(z, f2r[...]), g2r, b2r))
        z = _elu(ln(hidot(z, f3r[...]), g3r, b3r))
        z = hidot(z, fwr[...]) + fbr[...][None, :]
        outr[...] = z[0:1, :]

    return pl.pallas_call(
        body,
        out_shape=jax.ShapeDtypeStruct((1, 6), jnp.float32),
        compiler_params=_TC_PARAMS,
    )(h0, h1, h2, h3, att_wT, fc1_W, ln1_g, ln1_b, fc2_W, ln2_g, ln2_b,
      fc3_W, ln3_g, ln3_b, fco_W, fco_b)


def kernel(x, adj_or_data, W0, b0, g0, be0, W1, b1, g1, be1, W2, b2, g2, be2,
           W3, b3, g3, be3, att_w, fc1_W, ln1_g, ln1_b, fc2_W, ln2_g, ln2_b,
           fc3_W, ln3_g, ln3_b, fco_W, fco_b):
    src = adj_or_data[0]
    dst = adj_or_data[1]
    padi = jnp.full((EPAD - E,), N, dtype=jnp.int32)
    srcp = jnp.concatenate([src, padi]).reshape(NW, NCH, CH)
    dstp = jnp.concatenate([dst, padi]).reshape(NW, NCH, CH)
    ones128 = jnp.ones((CH, 128), jnp.float32)
    z128 = jnp.zeros((NPAD, 128), jnp.float32)
    xp = jnp.pad(x, ((0, NPAD - N), (0, 5)))
    W0p = jnp.pad(W0, ((0, 5), (0, 64)))
    W1p = jnp.pad(W1, ((0, 64), (0, 0)))
    W3p = jnp.pad(W3, ((0, 0), (0, 64)))
    pad64 = lambda v: jnp.pad(v, (0, 64))

    degp = _deg_call(dstp, ones128, z128)
    hs1, dinv = _tc_a(xp, W0p, degp)

    p1 = _agg_call(hs1, srcp, dstp, z128)
    h0, hs2 = _tc_combine(p1, hs1, dinv, pad64(b0), pad64(g0), pad64(be0),
                          W_next=W1p)

    p2 = _agg_call(hs2, srcp, dstp, z128)
    h1, hs3 = _tc_combine(p2, hs2, dinv, b1, g1, be1, W_next=W2)

    p3 = _agg_call(hs3, srcp, dstp, z128)
    h2, hs4 = _tc_combine(p3, hs3, dinv, b2, g2, be2, W_next=W3p, h_res=h1)

    p4 = _agg_call(hs4, srcp, dstp, z128)
    (h3,) = _tc_combine(p4, hs4, dinv, pad64(b3), pad64(g3), pad64(be3))

    return _tc_head(h0, h1, h2, h3, jnp.pad(att_w, ((0, 0), (0, 127))), fc1_W,
                    ln1_g, ln1_b, fc2_W, ln2_g, ln2_b,
                    fc3_W, ln3_g, ln3_b, fco_W, fco_b)
